# speedup vs baseline: 5.4135x; 5.4135x over previous
import functools

import jax
import jax.numpy as jnp
from jax import lax
from jax.experimental import pallas as pl
from jax.experimental.pallas import tpu as pltpu
from jax.experimental.pallas import tpu_sc as plsc

_N_NODES = 10000
_N_EDGES = 320000
_HID = 128
_EMB = 16

_NC = 2
_NS = 16
_NW = _NC * _NS
_EPT = _N_EDGES // _NW
_B = 80
_NCH = _EPT // _B


def _proj_body(x_ref, w1d_ref, w1s_ref, b1_ref, pd_ref, ps_ref):
    x = x_ref[...]
    pd_ref[...] = jnp.dot(x, w1d_ref[...], preferred_element_type=jnp.float32) + b1_ref[...]
    ps_ref[...] = jnp.dot(x, w1s_ref[...], preferred_element_type=jnp.float32)


def _proj(x, w1d, w1s, b1_row):
    return pl.pallas_call(
        _proj_body,
        out_shape=(
            jax.ShapeDtypeStruct((_N_NODES, _HID), jnp.float32),
            jax.ShapeDtypeStruct((_N_NODES, _HID), jnp.float32),
        ),
    )(x, w1d, w1s, b1_row)


_mesh = plsc.VectorSubcoreMesh(core_axis_name="c", subcore_axis_name="s")


@functools.partial(
    pl.kernel,
    out_type=jax.ShapeDtypeStruct((_N_EDGES, _HID), jnp.float32),
    mesh=_mesh,
    scratch_types=[
        pltpu.VMEM((_B,), jnp.int32),
        pltpu.VMEM((_B,), jnp.int32),
        pltpu.VMEM((_B, _HID), jnp.float32),
        pltpu.VMEM((_B, _HID), jnp.float32),
        pltpu.SemaphoreType.DMA,
        pltpu.SemaphoreType.DMA,
    ],
)
def _sc_edge(pd_hbm, ps_hbm, dst_hbm, src_hbm, h_hbm, dsti, srci, gd, gs, sem1, sem2):
    wid = lax.axis_index("s") * _NC + lax.axis_index("c")
    base0 = wid * _EPT

    def chunk(ci, carry):
        base = base0 + ci * _B
        pltpu.sync_copy(dst_hbm.at[pl.ds(base, _B)], dsti)
        pltpu.sync_copy(src_hbm.at[pl.ds(base, _B)], srci)
        cp1 = pltpu.async_copy(pd_hbm.at[dsti], gd, sem1)
        cp2 = pltpu.async_copy(ps_hbm.at[srci], gs, sem2)
        cp1.wait()
        cp2.wait()

        def row(i, c2):
            for c in range(_HID // 16):
                sl = pl.ds(c * 16, 16)
                gd[i, sl] = gd[i, sl] + gs[i, sl]
            return c2

        lax.fori_loop(0, _B, row, 0)
        pltpu.sync_copy(gd, h_hbm.at[pl.ds(base, _B)])
        return carry

    lax.fori_loop(0, _NCH, chunk, 0)


def _post_body(h_ref, dist_ref, wd_ref, w2_ref, b2_ref, out_ref):
    h = jnp.maximum(h_ref[...] + dist_ref[...] * wd_ref[...], 0.0)
    out = jnp.dot(h, w2_ref[...], preferred_element_type=jnp.float32) + b2_ref[...]
    out_ref[...] = jnp.maximum(out, 0.0)


def _post(h, dist_col, wd_row, w2, b2_row):
    be = 4000
    grid = _N_EDGES // be
    return pl.pallas_call(
        _post_body,
        grid=(grid,),
        in_specs=[
            pl.BlockSpec((be, _HID), lambda i: (i, 0)),
            pl.BlockSpec((be, 1), lambda i: (i, 0)),
            pl.BlockSpec((1, _HID), lambda i: (0, 0)),
            pl.BlockSpec((_HID, _EMB), lambda i: (0, 0)),
            pl.BlockSpec((1, _EMB), lambda i: (0, 0)),
        ],
        out_specs=pl.BlockSpec((be, _EMB), lambda i: (i, 0)),
        out_shape=jax.ShapeDtypeStruct((_N_EDGES, _EMB), jnp.float32),
    )(h, dist_col, wd_row, w2, b2_row)


def kernel(node_features, node_hidden_state, edge_index, distance, W1, b1, W2, b2):
    x = jnp.concatenate([node_features, node_hidden_state], axis=1)
    pd, ps = _proj(x, W1[0:130], W1[130:260], b1.reshape(1, _HID))
    h = _sc_edge(pd, ps, edge_index[1], edge_index[0])
    return _post(
        h,
        distance.reshape(_N_EDGES, 1),
        W1[260].reshape(1, _HID),
        W2,
        b2.reshape(1, _EMB),
    )
